# baseline (speedup 1.0000x reference)
import jax
import jax.numpy as jnp
from jax import lax
from jax.experimental import pallas as pl
from jax.experimental.pallas import tpu as pltpu
from jax.experimental.pallas import tpu_sc as plsc

N = 10000
E = 320000
D = 128
D_OUT = 12

NC = 2
NS = 16
NW = NC * NS
B = 128
NP = 10240
RPT = NP // NS
EPT = 10240
CH = EPT // B
E2 = NW * EPT

_mesh = plsc.VectorSubcoreMesh(core_axis_name="c", subcore_axis_name="s")


def _deg_body(dst_hbm, ones_hbm, zero_hbm, out_hbm, dst_all, ones_v, acc, sem):
    c = lax.axis_index("c")
    s = lax.axis_index("s")
    wid = s * NC + c
    base = s * RPT
    ebase = wid * EPT
    pltpu.sync_copy(zero_hbm, acc.at[pl.ds(base, RPT)])
    pltpu.sync_copy(ones_hbm, ones_v)
    pltpu.sync_copy(dst_hbm.at[pl.ds(ebase, EPT)], dst_all)
    plsc.subcore_barrier()

    @pl.loop(0, CH)
    def _(j):
        pltpu.async_copy(ones_v, acc.at[dst_all.at[pl.ds(j * B, B)]], sem,
                         add=True)

    @pl.loop(0, CH)
    def _(j):
        pltpu.make_async_copy(ones_v, acc.at[dst_all.at[pl.ds(j * B, B)]],
                              sem).wait()

    plsc.subcore_barrier()
    pltpu.sync_copy(acc.at[pl.ds(base, RPT)], out_hbm.at[c, pl.ds(base, RPT)])


_deg_kernel = pl.kernel(
    _deg_body,
    out_type=jax.ShapeDtypeStruct((NC, NP, D), jnp.float32),
    mesh=_mesh,
    scratch_types=[
        pltpu.VMEM((EPT,), jnp.int32),
        pltpu.VMEM((B, D), jnp.float32),
        pltpu.VMEM_SHARED((NP, D), jnp.float32),
        pltpu.SemaphoreType.DMA,
    ],
)


CH0 = 160
CH1 = 0
assert NS * (CH0 + CH1) * B == E2


def _agg_body(g_hbm, idx_hbm, zero_hbm, out_hbm,
              i0, i1, i2, i3, rows0, rows1, acc,
              is0, is1, is2, is3, gs0, gs1, ss0, ss1):
    c = lax.axis_index("c")
    s = lax.axis_index("s")
    base = s * RPT

    def idx_ref(k):
        return idx_hbm.at[pl.ds(k * (2 * B), 2 * B)]

    def load_idx(k, slot, sem):
        pltpu.async_copy(idx_ref(k), slot, sem)

    def wait_idx(slot, sem):
        pltpu.make_async_copy(idx_ref(0), slot, sem).wait()

    def gather(slot, rows, sem):
        pltpu.async_copy(g_hbm.at[slot.at[pl.ds(0, B)]], rows, sem)

    def wait_gather(slot, rows, sem):
        pltpu.make_async_copy(g_hbm.at[slot.at[pl.ds(0, B)]], rows, sem).wait()

    def scatter(rows, slot, sem):
        pltpu.async_copy(rows, acc.at[slot.at[pl.ds(B, B)]], sem, add=True)

    def wait_scatter(rows, slot, sem):
        pltpu.make_async_copy(rows, acc.at[slot.at[pl.ds(B, B)]], sem).wait()

    def run_pipe(cbase, chl):
        load_idx(cbase, i0, is0)
        load_idx(cbase + 1, i1, is1)
        load_idx(cbase + 2, i2, is2)
        load_idx(cbase + 3, i3, is3)
        wait_idx(i0, is0)
        gather(i0, rows0, gs0)

        @pl.loop(0, chl, step=4)
        def _(j):
            wait_idx(i1, is1)
            gather(i1, rows1, gs1)
            wait_gather(i0, rows0, gs0)
            scatter(rows0, i0, ss0)
            wait_scatter(rows0, i0, ss0)
            wait_idx(i2, is2)
            gather(i2, rows0, gs0)
            load_idx(cbase + lax.rem(j + 4, chl), i0, is0)
            wait_gather(i1, rows1, gs1)
            scatter(rows1, i1, ss1)
            wait_scatter(rows1, i1, ss1)
            wait_idx(i3, is3)
            gather(i3, rows1, gs1)
            load_idx(cbase + lax.rem(j + 5, chl), i1, is1)
            wait_gather(i2, rows0, gs0)
            scatter(rows0, i2, ss0)
            wait_scatter(rows0, i2, ss0)
            wait_idx(i0, is0)
            gather(i0, rows0, gs0)
            load_idx(cbase + lax.rem(j + 6, chl), i2, is2)
            wait_gather(i3, rows1, gs1)
            scatter(rows1, i3, ss1)
            wait_scatter(rows1, i3, ss1)
            load_idx(cbase + lax.rem(j + 7, chl), i3, is3)

        wait_gather(i0, rows0, gs0)
        wait_idx(i1, is1)
        wait_idx(i2, is2)
        wait_idx(i3, is3)

    pltpu.sync_copy(zero_hbm, acc.at[pl.ds(base, RPT)])
    plsc.subcore_barrier()

    @pl.when(c == 0)
    def _():
        run_pipe(s * CH0, CH0)

    if CH1 > 0:
        @pl.when(c == 1)
        def _():
            run_pipe(NS * CH0 + s * CH1, CH1)

    plsc.subcore_barrier()
    pltpu.sync_copy(acc.at[pl.ds(base, RPT)], out_hbm.at[c, pl.ds(base, RPT)])


_agg_kernel = pl.kernel(
    _agg_body,
    out_type=jax.ShapeDtypeStruct((NC, NP, D), jnp.float32),
    mesh=_mesh,
    scratch_types=[
        pltpu.VMEM((2 * B,), jnp.int32),
        pltpu.VMEM((2 * B,), jnp.int32),
        pltpu.VMEM((2 * B,), jnp.int32),
        pltpu.VMEM((2 * B,), jnp.int32),
        pltpu.VMEM((B, D), jnp.float32),
        pltpu.VMEM((B, D), jnp.float32),
        pltpu.VMEM_SHARED((NP, D), jnp.float32),
        pltpu.SemaphoreType.DMA,
        pltpu.SemaphoreType.DMA,
        pltpu.SemaphoreType.DMA,
        pltpu.SemaphoreType.DMA,
        pltpu.SemaphoreType.DMA,
        pltpu.SemaphoreType.DMA,
        pltpu.SemaphoreType.DMA,
        pltpu.SemaphoreType.DMA,
    ],
)


_BLK = 2000
_GRID = N // _BLK


def _mm_body(x_ref, w_ref, o_ref):
    o_ref[...] = jnp.dot(x_ref[...], w_ref[...], preferred_element_type=jnp.float32)


def _scale_body(h_ref, d_ref, o_ref):
    deg = d_ref[0, :, 0:1] + d_ref[1, :, 0:1] + 1.0
    o_ref[...] = lax.rsqrt(deg) * h_ref[...]


def _layer_body(s_ref, g_ref, d_ref, w_ref, b_ref, o_ref):
    deg = d_ref[0, :, 0:1] + d_ref[1, :, 0:1] + 1.0
    dinv = lax.rsqrt(deg)
    xk = jnp.maximum(dinv * (s_ref[0] + s_ref[1] + g_ref[...]) + b_ref[...], 0.0)
    o_ref[...] = dinv * jnp.dot(xk, w_ref[...], preferred_element_type=jnp.float32)


def _final_body(s_ref, g_ref, d_ref, b_ref, wfc_ref, bfc_ref, o_ref, acc_ref):
    i = pl.program_id(0)

    @pl.when(i == 0)
    def _():
        acc_ref[...] = jnp.zeros_like(acc_ref)

    deg = d_ref[0, :, 0:1] + d_ref[1, :, 0:1] + 1.0
    dinv = lax.rsqrt(deg)
    xk = jnp.maximum(dinv * (s_ref[0] + s_ref[1] + g_ref[...]) + b_ref[...], 0.0)
    acc_ref[...] += jnp.sum(xk, axis=0, keepdims=True)

    @pl.when(i == _GRID - 1)
    def _():
        o_ref[...] = (
            jnp.dot(acc_ref[...] * (1.0 / N), wfc_ref[...],
                    preferred_element_type=jnp.float32)
            + bfc_ref[...]
        )


def _row_spec():
    return pl.BlockSpec((_BLK, D), lambda i: (i, 0))


def _part_spec():
    return pl.BlockSpec((NC, _BLK, D), lambda i: (0, i, 0))


def _full(shape):
    return pl.BlockSpec(shape, lambda i: tuple(0 for _ in shape))


_matmul = pl.pallas_call(
    _mm_body,
    grid=(_GRID,),
    in_specs=[_row_spec(), _full((D, D))],
    out_specs=_row_spec(),
    out_shape=jax.ShapeDtypeStruct((N, D), jnp.float32),
)

_scale = pl.pallas_call(
    _scale_body,
    grid=(_GRID,),
    in_specs=[_row_spec(), _part_spec()],
    out_specs=_row_spec(),
    out_shape=jax.ShapeDtypeStruct((N, D), jnp.float32),
)

_layer = pl.pallas_call(
    _layer_body,
    grid=(_GRID,),
    in_specs=[_part_spec(), _row_spec(), _part_spec(), _full((D, D)),
              _full((1, D))],
    out_specs=_row_spec(),
    out_shape=jax.ShapeDtypeStruct((N, D), jnp.float32),
)

_final = pl.pallas_call(
    _final_body,
    grid=(_GRID,),
    in_specs=[_part_spec(), _row_spec(), _part_spec(), _full((1, D)),
              _full((D, D_OUT)), _full((1, D_OUT))],
    out_specs=_full((1, D_OUT)),
    out_shape=jax.ShapeDtypeStruct((1, D_OUT), jnp.float32),
    scratch_shapes=[pltpu.VMEM((1, D), jnp.float32)],
)


def kernel(x, edge_index, W1, b1, W2, b2, W3, b3, Wfc, bfc):
    pad = E2 - E
    src_p = jnp.concatenate([edge_index[0], jnp.zeros((pad,), jnp.int32)])
    dst_p = jnp.concatenate([edge_index[1], jnp.full((pad,), NP - 1, jnp.int32)])
    idx2 = jnp.stack([src_p.reshape(E2 // B, B), dst_p.reshape(E2 // B, B)],
                     axis=1).reshape(-1)
    ones_hbm = jnp.ones((B, D), jnp.float32)
    zeroD = jnp.zeros((RPT, D), jnp.float32)

    degp = _deg_kernel(dst_p, ones_hbm, zeroD)
    h1 = _matmul(x, W1)
    g1 = _scale(h1, degp)
    s1 = _agg_kernel(g1, idx2, zeroD)
    g2 = _layer(s1, g1, degp, W2, b1.reshape(1, D))
    s2 = _agg_kernel(g2, idx2, zeroD)
    g3 = _layer(s2, g2, degp, W3, b2.reshape(1, D))
    s3 = _agg_kernel(g3, idx2, zeroD)
    return _final(s3, g3, degp, b3.reshape(1, D), Wfc, bfc.reshape(1, D_OUT))


# speedup vs baseline: 1.1867x; 1.1867x over previous
import jax
import jax.numpy as jnp
from jax import lax
from jax.experimental import pallas as pl
from jax.experimental.pallas import tpu as pltpu
from jax.experimental.pallas import tpu_sc as plsc

N = 10000
E = 320000
D = 128
D_OUT = 12

NC = 2
NS = 16
NW = NC * NS
B = 128
NP = 10240
RPT = NP // NS
EPT = 10240
CH = EPT // B
E2 = NW * EPT

_mesh = plsc.VectorSubcoreMesh(core_axis_name="c", subcore_axis_name="s")


def _deg_body(dst_hbm, ones_hbm, zero_hbm, out_hbm, dst_all, ones_v, acc, sem):
    c = lax.axis_index("c")
    s = lax.axis_index("s")
    wid = s * NC + c
    base = s * RPT
    ebase = wid * EPT
    pltpu.sync_copy(zero_hbm, acc.at[pl.ds(base, RPT)])
    pltpu.sync_copy(ones_hbm, ones_v)
    pltpu.sync_copy(dst_hbm.at[pl.ds(ebase, EPT)], dst_all)
    plsc.subcore_barrier()

    @pl.loop(0, CH)
    def _(j):
        pltpu.async_copy(ones_v, acc.at[dst_all.at[pl.ds(j * B, B)]], sem,
                         add=True)

    @pl.loop(0, CH)
    def _(j):
        pltpu.make_async_copy(ones_v, acc.at[dst_all.at[pl.ds(j * B, B)]],
                              sem).wait()

    plsc.subcore_barrier()
    pltpu.sync_copy(acc.at[pl.ds(base, RPT)], out_hbm.at[c, pl.ds(base, RPT)])


_deg_kernel = pl.kernel(
    _deg_body,
    out_type=jax.ShapeDtypeStruct((NC, NP, D), jnp.float32),
    mesh=_mesh,
    scratch_types=[
        pltpu.VMEM((EPT,), jnp.int32),
        pltpu.VMEM((B, D), jnp.float32),
        pltpu.VMEM_SHARED((NP, D), jnp.float32),
        pltpu.SemaphoreType.DMA,
    ],
)


CH0 = 112
CH1 = 48
assert NS * (CH0 + CH1) * B == E2


def _agg_body(g_hbm, idx_hbm, zero_hbm, out_hbm,
              i0, i1, i2, i3, rows0, rows1, acc,
              is0, is1, is2, is3, gs0, gs1, ss0, ss1):
    c = lax.axis_index("c")
    s = lax.axis_index("s")
    base = s * RPT

    def idx_ref(k):
        return idx_hbm.at[pl.ds(k * (2 * B), 2 * B)]

    def load_idx(k, slot, sem):
        pltpu.async_copy(idx_ref(k), slot, sem)

    def wait_idx(slot, sem):
        pltpu.make_async_copy(idx_ref(0), slot, sem).wait()

    def gather(slot, rows, sem):
        pltpu.async_copy(g_hbm.at[slot.at[pl.ds(0, B)]], rows, sem)

    def wait_gather(slot, rows, sem):
        pltpu.make_async_copy(g_hbm.at[slot.at[pl.ds(0, B)]], rows, sem).wait()

    def scatter(rows, slot, sem):
        pltpu.async_copy(rows, acc.at[slot.at[pl.ds(B, B)]], sem, add=True)

    def wait_scatter(rows, slot, sem):
        pltpu.make_async_copy(rows, acc.at[slot.at[pl.ds(B, B)]], sem).wait()

    def run_pipe(cbase, chl):
        load_idx(cbase, i0, is0)
        load_idx(cbase + 1, i1, is1)
        load_idx(cbase + 2, i2, is2)
        load_idx(cbase + 3, i3, is3)
        wait_idx(i0, is0)
        gather(i0, rows0, gs0)

        @pl.loop(0, chl, step=4)
        def _(j):
            wait_idx(i1, is1)
            gather(i1, rows1, gs1)
            wait_gather(i0, rows0, gs0)
            scatter(rows0, i0, ss0)
            wait_scatter(rows0, i0, ss0)
            wait_idx(i2, is2)
            gather(i2, rows0, gs0)
            load_idx(cbase + lax.rem(j + 4, chl), i0, is0)
            wait_gather(i1, rows1, gs1)
            scatter(rows1, i1, ss1)
            wait_scatter(rows1, i1, ss1)
            wait_idx(i3, is3)
            gather(i3, rows1, gs1)
            load_idx(cbase + lax.rem(j + 5, chl), i1, is1)
            wait_gather(i2, rows0, gs0)
            scatter(rows0, i2, ss0)
            wait_scatter(rows0, i2, ss0)
            wait_idx(i0, is0)
            gather(i0, rows0, gs0)
            load_idx(cbase + lax.rem(j + 6, chl), i2, is2)
            wait_gather(i3, rows1, gs1)
            scatter(rows1, i3, ss1)
            wait_scatter(rows1, i3, ss1)
            load_idx(cbase + lax.rem(j + 7, chl), i3, is3)

        wait_gather(i0, rows0, gs0)
        wait_idx(i1, is1)
        wait_idx(i2, is2)
        wait_idx(i3, is3)

    pltpu.sync_copy(zero_hbm, acc.at[pl.ds(base, RPT)])
    plsc.subcore_barrier()

    @pl.when(c == 0)
    def _():
        run_pipe(s * CH0, CH0)

    if CH1 > 0:
        @pl.when(c == 1)
        def _():
            run_pipe(NS * CH0 + s * CH1, CH1)

    plsc.subcore_barrier()
    pltpu.sync_copy(acc.at[pl.ds(base, RPT)], out_hbm.at[c, pl.ds(base, RPT)])


_agg_kernel = pl.kernel(
    _agg_body,
    out_type=jax.ShapeDtypeStruct((NC, NP, D), jnp.float32),
    mesh=_mesh,
    scratch_types=[
        pltpu.VMEM((2 * B,), jnp.int32),
        pltpu.VMEM((2 * B,), jnp.int32),
        pltpu.VMEM((2 * B,), jnp.int32),
        pltpu.VMEM((2 * B,), jnp.int32),
        pltpu.VMEM((B, D), jnp.float32),
        pltpu.VMEM((B, D), jnp.float32),
        pltpu.VMEM_SHARED((NP, D), jnp.float32),
        pltpu.SemaphoreType.DMA,
        pltpu.SemaphoreType.DMA,
        pltpu.SemaphoreType.DMA,
        pltpu.SemaphoreType.DMA,
        pltpu.SemaphoreType.DMA,
        pltpu.SemaphoreType.DMA,
        pltpu.SemaphoreType.DMA,
        pltpu.SemaphoreType.DMA,
    ],
)


_BLK = 2000
_GRID = N // _BLK


def _mm_body(x_ref, w_ref, o_ref):
    o_ref[...] = jnp.dot(x_ref[...], w_ref[...], preferred_element_type=jnp.float32)


def _scale_body(h_ref, d_ref, o_ref):
    deg = d_ref[0, :, 0:1] + d_ref[1, :, 0:1] + 1.0
    o_ref[...] = lax.rsqrt(deg) * h_ref[...]


def _layer_body(s_ref, g_ref, d_ref, w_ref, b_ref, o_ref):
    deg = d_ref[0, :, 0:1] + d_ref[1, :, 0:1] + 1.0
    dinv = lax.rsqrt(deg)
    xk = jnp.maximum(dinv * (s_ref[0] + s_ref[1] + g_ref[...]) + b_ref[...], 0.0)
    o_ref[...] = dinv * jnp.dot(xk, w_ref[...], preferred_element_type=jnp.float32)


def _final_body(s_ref, g_ref, d_ref, b_ref, wfc_ref, bfc_ref, o_ref, acc_ref):
    i = pl.program_id(0)

    @pl.when(i == 0)
    def _():
        acc_ref[...] = jnp.zeros_like(acc_ref)

    deg = d_ref[0, :, 0:1] + d_ref[1, :, 0:1] + 1.0
    dinv = lax.rsqrt(deg)
    xk = jnp.maximum(dinv * (s_ref[0] + s_ref[1] + g_ref[...]) + b_ref[...], 0.0)
    acc_ref[...] += jnp.sum(xk, axis=0, keepdims=True)

    @pl.when(i == _GRID - 1)
    def _():
        o_ref[...] = (
            jnp.dot(acc_ref[...] * (1.0 / N), wfc_ref[...],
                    preferred_element_type=jnp.float32)
            + bfc_ref[...]
        )


def _row_spec():
    return pl.BlockSpec((_BLK, D), lambda i: (i, 0))


def _part_spec():
    return pl.BlockSpec((NC, _BLK, D), lambda i: (0, i, 0))


def _full(shape):
    return pl.BlockSpec(shape, lambda i: tuple(0 for _ in shape))


_matmul = pl.pallas_call(
    _mm_body,
    grid=(_GRID,),
    in_specs=[_row_spec(), _full((D, D))],
    out_specs=_row_spec(),
    out_shape=jax.ShapeDtypeStruct((N, D), jnp.float32),
)

_scale = pl.pallas_call(
    _scale_body,
    grid=(_GRID,),
    in_specs=[_row_spec(), _part_spec()],
    out_specs=_row_spec(),
    out_shape=jax.ShapeDtypeStruct((N, D), jnp.float32),
)

_layer = pl.pallas_call(
    _layer_body,
    grid=(_GRID,),
    in_specs=[_part_spec(), _row_spec(), _part_spec(), _full((D, D)),
              _full((1, D))],
    out_specs=_row_spec(),
    out_shape=jax.ShapeDtypeStruct((N, D), jnp.float32),
)

_final = pl.pallas_call(
    _final_body,
    grid=(_GRID,),
    in_specs=[_part_spec(), _row_spec(), _part_spec(), _full((1, D)),
              _full((D, D_OUT)), _full((1, D_OUT))],
    out_specs=_full((1, D_OUT)),
    out_shape=jax.ShapeDtypeStruct((1, D_OUT), jnp.float32),
    scratch_shapes=[pltpu.VMEM((1, D), jnp.float32)],
)


def kernel(x, edge_index, W1, b1, W2, b2, W3, b3, Wfc, bfc):
    pad = E2 - E
    src_p = jnp.concatenate([edge_index[0], jnp.zeros((pad,), jnp.int32)])
    dst_p = jnp.concatenate([edge_index[1], jnp.full((pad,), NP - 1, jnp.int32)])
    idx2 = jnp.stack([src_p.reshape(E2 // B, B), dst_p.reshape(E2 // B, B)],
                     axis=1).reshape(-1)
    ones_hbm = jnp.ones((B, D), jnp.float32)
    zeroD = jnp.zeros((RPT, D), jnp.float32)

    degp = _deg_kernel(dst_p, ones_hbm, zeroD)
    h1 = _matmul(x, W1)
    g1 = _scale(h1, degp)
    s1 = _agg_kernel(g1, idx2, zeroD)
    g2 = _layer(s1, g1, degp, W2, b1.reshape(1, D))
    s2 = _agg_kernel(g2, idx2, zeroD)
    g3 = _layer(s2, g2, degp, W3, b2.reshape(1, D))
    s3 = _agg_kernel(g3, idx2, zeroD)
    return _final(s3, g3, degp, b3.reshape(1, D), Wfc, bfc.reshape(1, D_OUT))
